# baseline (speedup 1.0000x reference)
import functools

import jax
import jax.numpy as jnp
from jax import lax
from jax.experimental import pallas as pl
from jax.experimental.pallas import tpu as pltpu
from jax.experimental.pallas import tpu_sc as plsc

_V = 63
_D = 64
_VPAD = 64
_LANES = 16


def _table_body(sym_ref, w_ref, b_ref, out_ref):
    out_ref[...] = (
        jnp.dot(sym_ref[...], w_ref[...], preferred_element_type=jnp.float32)
        + b_ref[0:1, :]
    )


def _fused_table(symbols, W, b):
    sym = jnp.pad(symbols, ((0, _VPAD - symbols.shape[0]), (0, 0)))
    b2 = jnp.broadcast_to(b.reshape(1, -1), (8, _D))
    return pl.pallas_call(
        _table_body,
        out_shape=jax.ShapeDtypeStruct((_VPAD, _D), jnp.float32),
    )(sym, W, b2)


@functools.cache
def _make_gather(bsz, seq, d):
    info = plsc.get_sparse_core_info()
    nc, ns = info.num_cores, info.num_subcores
    nw = nc * ns
    rows_w = bsz // nw
    gdim, rdim = d // 8, 8
    n_cb = rows_w // _LANES
    per_w = rows_w * seq
    nit = seq // 2
    mesh = plsc.VectorSubcoreMesh(core_axis_name="c", subcore_axis_name="s")

    @functools.partial(
        pl.kernel,
        out_type=jax.ShapeDtypeStruct((seq, gdim, nw, rdim, 128), jnp.float32),
        mesh=mesh,
        scratch_types=[
            pltpu.VMEM((per_w,), jnp.int32),
            pltpu.VMEM((_VPAD * d,), jnp.float32),
            pltpu.VMEM((gdim, rdim, 128), jnp.float32),
            pltpu.VMEM((gdim, rdim, 128), jnp.float32),
            pltpu.SemaphoreType.DMA,
            pltpu.SemaphoreType.DMA,
        ],
        compiler_params=pltpu.CompilerParams(use_tc_tiling_on_sc=False,
                                             needs_layout_passes=False),
    )
    def gather(table_hbm, qr_hbm, out_hbm, idx_all, table_v, st0, st1,
               os0, os1):
        wid = lax.axis_index("s") * nc + lax.axis_index("c")
        pltpu.sync_copy(table_hbm, table_v)
        pltpu.sync_copy(qr_hbm.at[pl.ds(wid * per_w, per_w)], idx_all)
        iota_seq = lax.iota(jnp.int32, _LANES) * seq
        slots = ((st0, os0), (st1, os1))

        def body(l2, carry):
            for s, (st, osem) in enumerate(slots):
                l = 2 * l2 + s

                @pl.when(l2 > 0)
                def _():
                    pltpu.make_async_copy(st, out_hbm.at[0, :, wid],
                                          osem).wait()

                def cbody(cb, carry2):
                    addr = iota_seq + (cb * (_LANES * seq) + l)
                    idx = plsc.load_gather(idx_all, [addr])
                    base = idx * d
                    for dd in range(d):
                        val = plsc.load_gather(table_v, [base + dd])
                        st[dd // rdim, dd % rdim, pl.ds(cb * _LANES, _LANES)] \
                            = val
                    return carry2

                lax.fori_loop(0, n_cb, cbody, 0)
                pltpu.async_copy(st, out_hbm.at[l, :, wid], osem)
            return carry

        lax.fori_loop(0, nit, body, 0)
        for st, osem in slots:
            pltpu.make_async_copy(st, out_hbm.at[0, :, wid], osem).wait()

    return gather


def kernel(QR, symbols, W, b):
    bsz, seq = QR.shape
    table = _fused_table(symbols, W, b)
    qr1 = QR.reshape(-1).astype(jnp.int32)
    out5 = _make_gather(bsz, seq, _D)(table.reshape(-1), qr1)
    return out5.transpose(2, 4, 0, 1, 3).reshape(bsz, seq, _D)


# speedup vs baseline: 1.6460x; 1.6460x over previous
import functools

import jax
import jax.numpy as jnp
from jax import lax
from jax.experimental import pallas as pl
from jax.experimental.pallas import tpu as pltpu
from jax.experimental.pallas import tpu_sc as plsc

_V = 63
_D = 64
_VPAD = 64
_LANES = 16


def _table_body(sym_ref, w_ref, b_ref, out_ref):
    out_ref[...] = (
        jnp.dot(sym_ref[...], w_ref[...], preferred_element_type=jnp.float32)
        + b_ref[0:1, :]
    )


def _fused_table(symbols, W, b):
    sym = jnp.pad(symbols, ((0, _VPAD - symbols.shape[0]), (0, 0)))
    b2 = jnp.broadcast_to(b.reshape(1, -1), (8, _D))
    return pl.pallas_call(
        _table_body,
        out_shape=jax.ShapeDtypeStruct((_VPAD, _D), jnp.float32),
    )(sym, W, b2)


@functools.cache
def _make_gather(bsz, seq, d):
    info = plsc.get_sparse_core_info()
    nc, ns = info.num_cores, info.num_subcores
    nw = nc * ns
    rows_w = bsz // nw
    gdim, rdim = d // 8, 8
    n_cb = rows_w // _LANES
    per_w = rows_w * seq
    nit = seq // 2
    mesh = plsc.VectorSubcoreMesh(core_axis_name="c", subcore_axis_name="s")

    @functools.partial(
        pl.kernel,
        out_type=jax.ShapeDtypeStruct((seq, gdim, nw, rdim, 128), jnp.float32),
        mesh=mesh,
        scratch_types=[
            pltpu.VMEM((per_w,), jnp.int32),
            pltpu.VMEM((_VPAD * d,), jnp.float32),
            pltpu.VMEM((gdim, rdim, 128), jnp.float32),
            pltpu.VMEM((gdim, rdim, 128), jnp.float32),
            pltpu.SemaphoreType.DMA,
            pltpu.SemaphoreType.DMA,
        ],
        compiler_params=pltpu.CompilerParams(use_tc_tiling_on_sc=False,
                                             needs_layout_passes=False),
    )
    def gather(table_hbm, qr_hbm, out_hbm, idx_all, table_v, st0, st1,
               os0, os1):
        wid = lax.axis_index("s") * nc + lax.axis_index("c")
        pltpu.sync_copy(table_hbm, table_v)
        pltpu.sync_copy(qr_hbm.at[pl.ds(wid * per_w, per_w)], idx_all)
        iota_seq = lax.iota(jnp.int32, _LANES) * seq
        slots = ((st0, os0), (st1, os1))

        def body(l2, carry):
            for s, (st, osem) in enumerate(slots):
                l = 2 * l2 + s

                @pl.when(l2 > 0)
                def _():
                    pltpu.make_async_copy(st, out_hbm.at[0, :, wid],
                                          osem).wait()

                @plsc.parallel_loop(0, n_cb, 1)
                def cbody(cb):
                    addr = iota_seq + (cb * (_LANES * seq) + l)
                    idx = plsc.load_gather(idx_all, [addr])
                    base = idx * d
                    for dd in range(d):
                        val = plsc.load_gather(table_v, [base + dd])
                        st[dd // rdim, dd % rdim, pl.ds(cb * _LANES, _LANES)] \
                            = val
                pltpu.async_copy(st, out_hbm.at[l, :, wid], osem)
            return carry

        lax.fori_loop(0, nit, body, 0)
        for st, osem in slots:
            pltpu.make_async_copy(st, out_hbm.at[0, :, wid], osem).wait()

    return gather


def kernel(QR, symbols, W, b):
    bsz, seq = QR.shape
    table = _fused_table(symbols, W, b)
    qr1 = QR.reshape(-1).astype(jnp.int32)
    out5 = _make_gather(bsz, seq, _D)(table.reshape(-1), qr1)
    return out5.transpose(2, 4, 0, 1, 3).reshape(bsz, seq, _D)


# speedup vs baseline: 7.6077x; 4.6219x over previous
import functools

import jax
import jax.numpy as jnp
from jax import lax
from jax.experimental import pallas as pl
from jax.experimental.pallas import tpu as pltpu
from jax.experimental.pallas import tpu_sc as plsc

_V = 63
_D = 64
_VPAD = 64
_LANES = 16


def _table_body(sym_ref, w_ref, b_ref, out_ref):
    out_ref[...] = (
        jnp.dot(sym_ref[...], w_ref[...], preferred_element_type=jnp.float32)
        + b_ref[0:1, :]
    )


def _fused_table(symbols, W, b):
    sym = jnp.pad(symbols, ((0, _VPAD - symbols.shape[0]), (0, 0)))
    b2 = jnp.broadcast_to(b.reshape(1, -1), (8, _D))
    return pl.pallas_call(
        _table_body,
        out_shape=jax.ShapeDtypeStruct((_VPAD, _D), jnp.float32),
    )(sym, W, b2)


@functools.cache
def _make_gather(bsz, seq, d):
    info = plsc.get_sparse_core_info()
    nc, ns = info.num_cores, info.num_subcores
    nw = nc * ns
    rows_w = bsz // nw
    gdim, rdim = d // 8, 8
    n_cb = rows_w // _LANES
    per_w = rows_w * seq
    nit = seq // 2
    mesh = plsc.VectorSubcoreMesh(core_axis_name="c", subcore_axis_name="s")

    @functools.partial(
        pl.kernel,
        out_type=jax.ShapeDtypeStruct((seq, gdim, nw, rdim, 128), jnp.float32),
        mesh=mesh,
        scratch_types=[
            pltpu.VMEM((per_w,), jnp.int32),
            pltpu.VMEM((_VPAD * (d + 1),), jnp.float32),
            pltpu.VMEM((gdim, rdim, 128), jnp.float32),
            pltpu.VMEM((gdim, rdim, 128), jnp.float32),
            pltpu.SemaphoreType.DMA,
            pltpu.SemaphoreType.DMA,
        ],
        compiler_params=pltpu.CompilerParams(use_tc_tiling_on_sc=False,
                                             needs_layout_passes=False),
    )
    def gather(table_hbm, qr_hbm, out_hbm, idx_all, table_v, st0, st1,
               os0, os1):
        wid = lax.axis_index("s") * nc + lax.axis_index("c")
        pltpu.sync_copy(table_hbm, table_v)
        pltpu.sync_copy(qr_hbm.at[pl.ds(wid * per_w, per_w)], idx_all)
        iota_seq = lax.iota(jnp.int32, _LANES) * seq
        slots = ((st0, os0), (st1, os1))

        def body(l2, carry):
            for s, (st, osem) in enumerate(slots):
                l = 2 * l2 + s

                @pl.when(l2 > 0)
                def _():
                    pltpu.make_async_copy(st, out_hbm.at[0, :, wid],
                                          osem).wait()

                @plsc.parallel_loop(0, n_cb, 1)
                def cbody(cb):
                    addr = iota_seq + (cb * (_LANES * seq) + l)
                    idx = plsc.load_gather(idx_all, [addr])
                    base = idx * (d + 1)
                    for dd in range(d):
                        val = plsc.load_gather(table_v, [base + dd])
                        st[dd // rdim, dd % rdim, pl.ds(cb * _LANES, _LANES)] \
                            = val
                pltpu.async_copy(st, out_hbm.at[l, :, wid], osem)
            return carry

        lax.fori_loop(0, nit, body, 0)
        for st, osem in slots:
            pltpu.make_async_copy(st, out_hbm.at[0, :, wid], osem).wait()

    return gather


def kernel(QR, symbols, W, b):
    bsz, seq = QR.shape
    table = _fused_table(symbols, W, b)
    tpad = jnp.pad(table, ((0, 0), (0, 1)))
    qr1 = QR.reshape(-1).astype(jnp.int32)
    out5 = _make_gather(bsz, seq, _D)(tpad.reshape(-1), qr1)
    return out5.transpose(2, 4, 0, 1, 3).reshape(bsz, seq, _D)
